# Initial kernel scaffold; baseline (speedup 1.0000x reference)
import jax
import jax.numpy as jnp
from jax.experimental import pallas as pl


def kernel(node_embeddings, rel_unary_idx, rel_binary_idx, rel_ternary_idx, W1_inner, b1_inner, W1_outer, b1_outer, W2_inner, b2_inner, W2_outer, b2_outer, W3_inner, b3_inner, W3_outer, b3_outer):
    raise NotImplementedError("write your pallas kernel here")


# speedup vs baseline: 1.1065x; 1.1065x over previous
import functools

import jax
import jax.numpy as jnp
from jax import lax
from jax.experimental import pallas as pl
from jax.experimental.pallas import tpu as pltpu
from jax.experimental.pallas import tpu_sc as plsc

EMB = 128
NC, NS = 2, 16
NW = NC * NS
CHUNK = 128


def _pad_rows(n):
    m = NW * CHUNK
    return ((n + m - 1) // m) * m


def _sc_gather(table, idx_list):
    nseg = len(idx_list)
    mesh = plsc.VectorSubcoreMesh(core_axis_name="c", subcore_axis_name="s")
    out_type = [jax.ShapeDtypeStruct((idx.shape[0], EMB), jnp.float32)
                for idx in idx_list]

    @functools.partial(
        pl.kernel,
        out_type=out_type,
        mesh=mesh,
        scratch_types=[
            pltpu.VMEM((CHUNK,), jnp.int32),
            pltpu.VMEM((CHUNK, EMB), jnp.float32),
            pltpu.SemaphoreType.DMA,
        ],
    )
    def gather_k(table_hbm, *refs):
        idx_hbms = refs[:nseg]
        out_hbms = refs[nseg:2 * nseg]
        idx_v, rows_v, sem = refs[2 * nseg:]
        wid = lax.axis_index("s") * NC + lax.axis_index("c")

        for s in range(nseg):
            n = out_hbms[s].shape[0]
            chunks_w = n // (NW * CHUNK)
            base = wid * chunks_w * CHUNK

            def body(c, carry, idx_hbm=idx_hbms[s], out_hbm=out_hbms[s],
                     base=base):
                off = base + c * CHUNK
                pltpu.sync_copy(idx_hbm.at[pl.ds(off, CHUNK)], idx_v)
                pltpu.async_copy(table_hbm.at[idx_v], rows_v, sem).wait()
                pltpu.sync_copy(rows_v, out_hbm.at[pl.ds(off, CHUNK)])
                return carry

            lax.fori_loop(0, chunks_w, body, 0)

    return gather_k(table, *idx_list)


def _mish(x):
    return x * jnp.tanh(jax.nn.softplus(x))


def _mlp_block(arity, nt, tb, parts, wi_t, bi, wo_t, bo):
    d = arity * EMB

    def body(*refs):
        part_refs = refs[:arity]
        wi_ref, bi_ref, wo_ref, bo_ref = refs[arity:arity + 4]
        out_ref = refs[arity + 4]
        xs = [p[...] for p in part_refs]
        acc = bi_ref[...]
        for k in range(arity):
            acc = acc + jnp.dot(xs[k], wi_ref[k * EMB:(k + 1) * EMB, :],
                                preferred_element_type=jnp.float32)
        h = _mish(acc)
        o = jnp.dot(h, wo_ref[...], preferred_element_type=jnp.float32)
        o = o + bo_ref[...]
        for k in range(arity):
            out_ref[:, k, :] = xs[k] + o[:, k * EMB:(k + 1) * EMB]

    grid = nt // tb
    in_specs = (
        [pl.BlockSpec((tb, EMB), lambda i: (i, 0)) for _ in range(arity)]
        + [pl.BlockSpec((d, d), lambda i: (0, 0)),
           pl.BlockSpec((1, d), lambda i: (0, 0)),
           pl.BlockSpec((d, d), lambda i: (0, 0)),
           pl.BlockSpec((1, d), lambda i: (0, 0))]
    )
    return pl.pallas_call(
        body,
        grid=(grid,),
        in_specs=in_specs,
        out_specs=pl.BlockSpec((tb, arity, EMB), lambda i: (i, 0, 0)),
        out_shape=jax.ShapeDtypeStruct((nt, arity, EMB), jnp.float32),
        compiler_params=pltpu.CompilerParams(
            dimension_semantics=("arbitrary",)),
    )(*parts, wi_t, bi, wo_t, bo)


def _pad_idx(idx, n_pad):
    n = idx.shape[0]
    return jnp.pad(idx, (0, n_pad - n))


def kernel(node_embeddings, rel_unary_idx, rel_binary_idx, rel_ternary_idx,
           W1_inner, b1_inner, W1_outer, b1_outer,
           W2_inner, b2_inner, W2_outer, b2_outer,
           W3_inner, b3_inner, W3_outer, b3_outer):
    n1 = rel_unary_idx.shape[0]
    n2 = rel_binary_idx.shape[0] // 2
    n3 = rel_ternary_idx.shape[0] // 3

    i2 = rel_binary_idx.reshape(n2, 2)
    i3 = rel_ternary_idx.reshape(n3, 3)
    idx_list = [
        _pad_idx(rel_unary_idx, _pad_rows(n1)),
        _pad_idx(i2[:, 0], _pad_rows(n2)),
        _pad_idx(i2[:, 1], _pad_rows(n2)),
        _pad_idx(i3[:, 0], _pad_rows(n3)),
        _pad_idx(i3[:, 1], _pad_rows(n3)),
        _pad_idx(i3[:, 2], _pad_rows(n3)),
    ]
    g1, g2a, g2b, g3a, g3b, g3c = _sc_gather(node_embeddings, idx_list)

    o1 = _mlp_block(1, n1, 1000, [g1],
                    W1_inner.T, b1_inner.reshape(1, -1),
                    W1_outer.T, b1_outer.reshape(1, -1))
    o2 = _mlp_block(2, n2, 1000, [g2a, g2b],
                    W2_inner.T, b2_inner.reshape(1, -1),
                    W2_outer.T, b2_outer.reshape(1, -1))
    o3 = _mlp_block(3, n3, 1000, [g3a, g3b, g3c],
                    W3_inner.T, b3_inner.reshape(1, -1),
                    W3_outer.T, b3_outer.reshape(1, -1))

    output_messages = jnp.concatenate(
        [o1.reshape(-1, EMB), o2.reshape(-1, EMB), o3.reshape(-1, EMB)], axis=0)
    output_indices = jnp.concatenate(
        [rel_unary_idx, rel_binary_idx, rel_ternary_idx], axis=0)
    return (output_messages, output_indices)


# speedup vs baseline: 1.6341x; 1.4768x over previous
import functools

import jax
import jax.numpy as jnp
from jax import lax
from jax.experimental import pallas as pl
from jax.experimental.pallas import tpu as pltpu
from jax.experimental.pallas import tpu_sc as plsc

EMB = 128
NC, NS = 2, 16
NW = NC * NS
CHUNK = 128


NBUF = 5


def _sc_gather(table, idx_mat):
    cw = idx_mat.shape[1]
    n_chunks = NW * cw
    p = cw // NBUF
    assert cw % NBUF == 0
    mesh = plsc.VectorSubcoreMesh(core_axis_name="c", subcore_axis_name="s")

    @functools.partial(
        pl.kernel,
        out_type=jax.ShapeDtypeStruct((n_chunks * CHUNK, EMB), jnp.float32),
        mesh=mesh,
        scratch_types=(
            [pltpu.VMEM((cw, CHUNK), jnp.int32)]
            + [pltpu.VMEM((CHUNK, EMB), jnp.float32) for _ in range(NBUF)]
            + [pltpu.SemaphoreType.DMA for _ in range(2 * NBUF)]
        ),
    )
    def gather_k(table_hbm, idx_hbm, out_hbm, idx_v, *rest):
        rows = rest[:NBUF]
        gsem = rest[NBUF:2 * NBUF]
        wsem = rest[2 * NBUF:]
        wid = lax.axis_index("s") * NC + lax.axis_index("c")
        cbase = wid * cw
        rbase = cbase * CHUNK

        pltpu.sync_copy(idx_hbm.at[wid], idx_v)
        for b in range(NBUF):
            pltpu.async_copy(table_hbm.at[idx_v.at[b]], rows[b], gsem[b])

        def body(i, carry):
            for b in range(NBUF):
                c = i * NBUF + b
                pltpu.make_async_copy(table_hbm.at[idx_v.at[c]], rows[b],
                                      gsem[b]).wait()
                pltpu.async_copy(
                    rows[b], out_hbm.at[pl.ds(rbase + c * CHUNK, CHUNK)],
                    wsem[b])

            @pl.when(i < p - 1)
            def _():
                for b in range(NBUF):
                    c2 = (i + 1) * NBUF + b
                    pltpu.make_async_copy(
                        rows[b], out_hbm.at[pl.ds(rbase, CHUNK)],
                        wsem[b]).wait()
                    pltpu.async_copy(table_hbm.at[idx_v.at[c2]], rows[b],
                                     gsem[b])
            return carry

        lax.fori_loop(0, p, body, 0)
        for b in range(NBUF):
            pltpu.make_async_copy(rows[b], out_hbm.at[pl.ds(rbase, CHUNK)],
                                  wsem[b]).wait()

    return gather_k(table, idx_mat)


def _mish(x):
    return x * jnp.tanh(jax.nn.softplus(x))


def _mlp_block(arity, nt, tb, gathered, offs, wi_t, bi, wo_t, bo):
    d = arity * EMB

    def body(*refs):
        part_refs = refs[:arity]
        wi_ref, bi_ref, wo_ref, bo_ref = refs[arity:arity + 4]
        out_ref = refs[arity + 4]
        xs = [p[...] for p in part_refs]
        acc = bi_ref[...]
        for k in range(arity):
            acc = acc + jnp.dot(xs[k], wi_ref[k * EMB:(k + 1) * EMB, :],
                                preferred_element_type=jnp.float32)
        h = _mish(acc)
        o = jnp.dot(h, wo_ref[...], preferred_element_type=jnp.float32)
        o = o + bo_ref[...]
        for k in range(arity):
            out_ref[:, k, :] = xs[k] + o[:, k * EMB:(k + 1) * EMB]

    grid = nt // tb
    in_specs = (
        [pl.BlockSpec((tb, EMB), lambda i, o=off // tb: (o + i, 0))
         for off in offs]
        + [pl.BlockSpec((d, d), lambda i: (0, 0)),
           pl.BlockSpec((1, d), lambda i: (0, 0)),
           pl.BlockSpec((d, d), lambda i: (0, 0)),
           pl.BlockSpec((1, d), lambda i: (0, 0))]
    )
    return pl.pallas_call(
        body,
        grid=(grid,),
        in_specs=in_specs,
        out_specs=pl.BlockSpec((tb, arity, EMB), lambda i: (i, 0, 0)),
        out_shape=jax.ShapeDtypeStruct((nt, arity, EMB), jnp.float32),
        compiler_params=pltpu.CompilerParams(
            dimension_semantics=("arbitrary",)),
    )(*([gathered] * arity), wi_t, bi, wo_t, bo)


def kernel(node_embeddings, rel_unary_idx, rel_binary_idx, rel_ternary_idx,
           W1_inner, b1_inner, W1_outer, b1_outer,
           W2_inner, b2_inner, W2_outer, b2_outer,
           W3_inner, b3_inner, W3_outer, b3_outer):
    n1 = rel_unary_idx.shape[0]
    n2 = rel_binary_idx.shape[0] // 2
    n3 = rel_ternary_idx.shape[0] // 3

    i2 = rel_binary_idx.reshape(n2, 2)
    i3 = rel_ternary_idx.reshape(n3, 3)
    total = n1 + 2 * n2 + 3 * n3
    m = NW * CHUNK * NBUF
    total_pad = ((total + m - 1) // m) * m
    idx_flat = jnp.concatenate([
        rel_unary_idx, i2[:, 0], i2[:, 1], i3[:, 0], i3[:, 1], i3[:, 2],
        jnp.zeros((total_pad - total,), rel_unary_idx.dtype)])
    g = _sc_gather(node_embeddings, idx_flat.reshape(NW, -1, CHUNK))

    off1 = [0]
    off2 = [n1, n1 + n2]
    off3 = [n1 + 2 * n2, n1 + 2 * n2 + n3, n1 + 2 * n2 + 2 * n3]
    o1 = _mlp_block(1, n1, 1000, g, off1,
                    W1_inner.T, b1_inner.reshape(1, -1),
                    W1_outer.T, b1_outer.reshape(1, -1))
    o2 = _mlp_block(2, n2, 1000, g, off2,
                    W2_inner.T, b2_inner.reshape(1, -1),
                    W2_outer.T, b2_outer.reshape(1, -1))
    o3 = _mlp_block(3, n3, 1000, g, off3,
                    W3_inner.T, b3_inner.reshape(1, -1),
                    W3_outer.T, b3_outer.reshape(1, -1))

    output_messages = jnp.concatenate(
        [o1.reshape(-1, EMB), o2.reshape(-1, EMB), o3.reshape(-1, EMB)], axis=0)
    output_indices = jnp.concatenate(
        [rel_unary_idx, rel_binary_idx, rel_ternary_idx], axis=0)
    return (output_messages, output_indices)


# speedup vs baseline: 1.6981x; 1.0392x over previous
import functools

import jax
import jax.numpy as jnp
from jax import lax
from jax.experimental import pallas as pl
from jax.experimental.pallas import tpu as pltpu
from jax.experimental.pallas import tpu_sc as plsc

EMB = 128
NC, NS = 2, 16
NW = NC * NS
CHUNK = 128


NBUF = 5


def _sc_gather(table, idx_mat):
    cw = idx_mat.shape[1]
    n_chunks = NW * cw
    p = cw // NBUF
    assert cw % NBUF == 0
    mesh = plsc.VectorSubcoreMesh(core_axis_name="c", subcore_axis_name="s")

    @functools.partial(
        pl.kernel,
        out_type=jax.ShapeDtypeStruct((n_chunks * CHUNK, EMB), jnp.float32),
        mesh=mesh,
        scratch_types=(
            [pltpu.VMEM((cw, CHUNK), jnp.int32)]
            + [pltpu.VMEM((CHUNK, EMB), jnp.float32) for _ in range(NBUF)]
            + [pltpu.SemaphoreType.DMA for _ in range(2 * NBUF)]
        ),
    )
    def gather_k(table_hbm, idx_hbm, out_hbm, idx_v, *rest):
        rows = rest[:NBUF]
        gsem = rest[NBUF:2 * NBUF]
        wsem = rest[2 * NBUF:]
        wid = lax.axis_index("s") * NC + lax.axis_index("c")
        cbase = wid * cw
        rbase = cbase * CHUNK

        pltpu.sync_copy(idx_hbm.at[wid], idx_v)
        for b in range(NBUF):
            pltpu.async_copy(table_hbm.at[idx_v.at[b]], rows[b], gsem[b])

        def body(i, carry):
            for b in range(NBUF):
                c = i * NBUF + b
                pltpu.make_async_copy(table_hbm.at[idx_v.at[c]], rows[b],
                                      gsem[b]).wait()
                pltpu.async_copy(
                    rows[b], out_hbm.at[pl.ds(rbase + c * CHUNK, CHUNK)],
                    wsem[b])

            @pl.when(i < p - 1)
            def _():
                for b in range(NBUF):
                    c2 = (i + 1) * NBUF + b
                    pltpu.make_async_copy(
                        rows[b], out_hbm.at[pl.ds(rbase, CHUNK)],
                        wsem[b]).wait()
                    pltpu.async_copy(table_hbm.at[idx_v.at[c2]], rows[b],
                                     gsem[b])
            return carry

        lax.fori_loop(0, p, body, 0)
        for b in range(NBUF):
            pltpu.make_async_copy(rows[b], out_hbm.at[pl.ds(rbase, CHUNK)],
                                  wsem[b]).wait()

    return gather_k(table, idx_mat)


def _mish(x):
    u = jnp.exp(jnp.minimum(x, 30.0))
    v = u * (u + 2.0)
    return x * (v / (v + 2.0))


def _mlp_block(arity, nt, tb, gathered, offs, wi_t, bi, wo_t, bo):
    d = arity * EMB

    def body(*refs):
        part_refs = refs[:arity]
        wi_ref, bi_ref, wo_ref, bo_ref = refs[arity:arity + 4]
        out_ref = refs[arity + 4]
        xs = [p[...] for p in part_refs]
        acc = bi_ref[...]
        for k in range(arity):
            acc = acc + jnp.dot(xs[k], wi_ref[k * EMB:(k + 1) * EMB, :],
                                preferred_element_type=jnp.float32)
        h = _mish(acc)
        o = jnp.dot(h, wo_ref[...], preferred_element_type=jnp.float32)
        o = o + bo_ref[...]
        for k in range(arity):
            out_ref[:, k, :] = xs[k] + o[:, k * EMB:(k + 1) * EMB]

    grid = nt // tb
    in_specs = (
        [pl.BlockSpec((tb, EMB), lambda i, o=off // tb: (o + i, 0))
         for off in offs]
        + [pl.BlockSpec((d, d), lambda i: (0, 0)),
           pl.BlockSpec((1, d), lambda i: (0, 0)),
           pl.BlockSpec((d, d), lambda i: (0, 0)),
           pl.BlockSpec((1, d), lambda i: (0, 0))]
    )
    return pl.pallas_call(
        body,
        grid=(grid,),
        in_specs=in_specs,
        out_specs=pl.BlockSpec((tb, arity, EMB), lambda i: (i, 0, 0)),
        out_shape=jax.ShapeDtypeStruct((nt, arity, EMB), jnp.float32),
        compiler_params=pltpu.CompilerParams(
            dimension_semantics=("arbitrary",)),
    )(*([gathered] * arity), wi_t, bi, wo_t, bo)


def kernel(node_embeddings, rel_unary_idx, rel_binary_idx, rel_ternary_idx,
           W1_inner, b1_inner, W1_outer, b1_outer,
           W2_inner, b2_inner, W2_outer, b2_outer,
           W3_inner, b3_inner, W3_outer, b3_outer):
    n1 = rel_unary_idx.shape[0]
    n2 = rel_binary_idx.shape[0] // 2
    n3 = rel_ternary_idx.shape[0] // 3

    i2 = rel_binary_idx.reshape(n2, 2)
    i3 = rel_ternary_idx.reshape(n3, 3)
    total = n1 + 2 * n2 + 3 * n3
    m = NW * CHUNK * NBUF
    total_pad = ((total + m - 1) // m) * m
    idx_flat = jnp.concatenate([
        rel_unary_idx, i2[:, 0], i2[:, 1], i3[:, 0], i3[:, 1], i3[:, 2],
        jnp.zeros((total_pad - total,), rel_unary_idx.dtype)])
    g = _sc_gather(node_embeddings, idx_flat.reshape(NW, -1, CHUNK))

    off1 = [0]
    off2 = [n1, n1 + n2]
    off3 = [n1 + 2 * n2, n1 + 2 * n2 + n3, n1 + 2 * n2 + 2 * n3]
    o1 = _mlp_block(1, n1, 1000, g, off1,
                    W1_inner.T, b1_inner.reshape(1, -1),
                    W1_outer.T, b1_outer.reshape(1, -1))
    o2 = _mlp_block(2, n2, 1000, g, off2,
                    W2_inner.T, b2_inner.reshape(1, -1),
                    W2_outer.T, b2_outer.reshape(1, -1))
    o3 = _mlp_block(3, n3, 1000, g, off3,
                    W3_inner.T, b3_inner.reshape(1, -1),
                    W3_outer.T, b3_outer.reshape(1, -1))

    output_messages = jnp.concatenate(
        [o1.reshape(-1, EMB), o2.reshape(-1, EMB), o3.reshape(-1, EMB)], axis=0)
    output_indices = jnp.concatenate(
        [rel_unary_idx, rel_binary_idx, rel_ternary_idx], axis=0)
    return (output_messages, output_indices)
